# baseline (speedup 1.0000x reference)
import functools

import jax
import jax.numpy as jnp
from jax import lax
from jax.experimental import pallas as pl
from jax.experimental.pallas import tpu as pltpu
from jax.experimental.pallas import tpu_sc as plsc

_NEG = float("-inf")
_BIG = 1 << 30
NCAND = 128


def _rowmax_body(x_ref, bm_ref):
    bm_ref[...] = jnp.max(x_ref[...], axis=2)[:, None, :]


def _extract_topk(x, n_iter, vals0, pos0):
    B, L = x.shape
    iota = lax.broadcasted_iota(jnp.int32, (B, L), 1)
    slot = lax.broadcasted_iota(jnp.int32, (B, NCAND), 1)

    def step(it, carry):
        x_c, vals, pos = carry
        m = jnp.max(x_c, axis=1, keepdims=True)
        p = jnp.min(jnp.where(x_c == m, iota, _BIG), axis=1, keepdims=True)
        vals = jnp.where(slot == it, m, vals)
        pos = jnp.where(slot == it, p, pos)
        x_c = jnp.where(iota == p, _NEG, x_c)
        return x_c, vals, pos

    return lax.fori_loop(0, n_iter, step, (x, vals0, pos0))


def _select_body(n_iter, x_ref, vals_ref, pos_ref):
    x = x_ref[...]
    B = x.shape[0]
    vals0 = jnp.full((B, NCAND), _NEG, jnp.float32)
    pos0 = jnp.zeros((B, NCAND), jnp.int32)
    _, vals, pos = _extract_topk(x, n_iter, vals0, pos0)
    vals_ref[...] = vals
    pos_ref[...] = pos


def _final_body(n_iter, N, C, x_ref, bids_ref, vals_ref, pos_ref, gr_ref, gm_ref):
    x = x_ref[...]
    B = x.shape[0]
    vals0 = jnp.full((B, NCAND), _NEG, jnp.float32)
    pos0 = jnp.zeros((B, NCAND), jnp.int32)
    _, vals, pos = _extract_topk(x, n_iter, vals0, pos0)
    vals_ref[...] = vals
    pos_ref[...] = pos
    cand = pos // C
    bids = bids_ref[...]
    jj = lax.broadcasted_iota(jnp.int32, (B, NCAND, NCAND), 2)
    onehot = (cand[:, :, None] == jj).astype(jnp.int32)
    box_id = jnp.sum(onehot * bids[:, None, :], axis=2)
    row = lax.broadcasted_iota(jnp.int32, (B, NCAND), 0)
    gid = box_id + row * N
    gr = gid // 32
    gr_ref[...] = gr
    gm_ref[...] = (gid - gr * 32) * 4


def _topk_rows(x, n_iter):
    B, L = x.shape
    return pl.pallas_call(
        functools.partial(_select_body, n_iter),
        in_specs=[pl.BlockSpec((B, L), lambda: (0, 0))],
        out_specs=[
            pl.BlockSpec((B, NCAND), lambda: (0, 0)),
            pl.BlockSpec((B, NCAND), lambda: (0, 0)),
        ],
        out_shape=[
            jax.ShapeDtypeStruct((B, NCAND), jnp.float32),
            jax.ShapeDtypeStruct((B, NCAND), jnp.int32),
        ],
    )(x)


def _sc_postprocess(vals, pos, gr, bbox128, n_img, per_img, C):
    total = n_img * per_img
    per_w = total // 32

    @functools.partial(
        pl.kernel,
        mesh=plsc.VectorSubcoreMesh(core_axis_name="c", subcore_axis_name="s"),
        out_type=[
            jax.ShapeDtypeStruct((total,), jnp.float32),
            jax.ShapeDtypeStruct((total,), jnp.int32),
            jax.ShapeDtypeStruct((total, 128), jnp.float32),
        ],
        scratch_types=[
            pltpu.VMEM((per_w,), jnp.float32),
            pltpu.VMEM((per_w,), jnp.int32),
            pltpu.VMEM((per_w,), jnp.int32),
            pltpu.VMEM((per_w,), jnp.float32),
            pltpu.VMEM((per_w,), jnp.int32),
            pltpu.VMEM((per_w, 128), jnp.float32),
            pltpu.SemaphoreType.DMA,
        ],
    )
    def k(vals_hbm, pos_hbm, gr_hbm, bbox_hbm, sc_hbm, lb_hbm, st_hbm,
          vv, pp, grv, sc_v, lb_v, br, sem):
        wid = lax.axis_index("s") * 2 + lax.axis_index("c")
        base = wid * per_w
        pltpu.sync_copy(vals_hbm.at[pl.ds(base, per_w)], vv)
        pltpu.sync_copy(pos_hbm.at[pl.ds(base, per_w)], pp)
        pltpu.sync_copy(gr_hbm.at[pl.ds(base, per_w)], grv)
        for c in range(per_w // 16):
            sl = pl.ds(c * 16, 16)
            p = pp[sl]
            cand = lax.div(p, jnp.int32(C))
            lb_v[sl] = p - cand * C
            v = vv[sl]
            sc_v[sl] = 1.0 / (1.0 + jnp.exp(-v))
        pltpu.async_copy(bbox_hbm.at[grv], br, sem).wait()
        pltpu.sync_copy(sc_v, sc_hbm.at[pl.ds(base, per_w)])
        pltpu.sync_copy(lb_v, lb_hbm.at[pl.ds(base, per_w)])
        pltpu.sync_copy(br, st_hbm.at[pl.ds(base, per_w)])

    return k(vals, pos, gr, bbox128)


def _boxes_body(st_ref, gm_ref, scale_ref, out_ref):
    st = st_ref[...]
    gmv = gm_ref[...]
    sc = scale_ref[...]
    col = lax.broadcasted_iota(jnp.int32, st.shape, 1)
    x_c = jnp.sum(jnp.where(col == gmv, st, 0.0), axis=1)
    y_c = jnp.sum(jnp.where(col == gmv + 1, st, 0.0), axis=1)
    w = jnp.sum(jnp.where(col == gmv + 2, st, 0.0), axis=1)
    h = jnp.sum(jnp.where(col == gmv + 3, st, 0.0), axis=1)
    out_ref[...] = jnp.stack(
        [x_c - 0.5 * w, y_c - 0.5 * h, x_c + 0.5 * w, y_c + 0.5 * h], axis=1
    ) * sc


def kernel(out_logits, out_bbox, target_sizes):
    B, N, C = out_logits.shape

    bm = pl.pallas_call(
        _rowmax_body,
        grid=(B,),
        in_specs=[pl.BlockSpec((1, N, C), lambda b: (b, 0, 0))],
        out_specs=pl.BlockSpec((1, 1, N), lambda b: (b, 0, 0)),
        out_shape=jax.ShapeDtypeStruct((B, 1, N), jnp.float32),
    )(out_logits)
    bm = bm.reshape(B, N)

    _, bids = _topk_rows(bm, NCAND)
    bids = jnp.sort(bids, axis=1)

    g = jnp.take_along_axis(out_logits, bids[:, :, None], axis=1)

    vals, pos, gr, gm = pl.pallas_call(
        functools.partial(_final_body, 100, N, C),
        in_specs=[
            pl.BlockSpec((B, NCAND * C), lambda: (0, 0)),
            pl.BlockSpec((B, NCAND), lambda: (0, 0)),
        ],
        out_specs=[pl.BlockSpec((B, NCAND), lambda: (0, 0))] * 4,
        out_shape=[
            jax.ShapeDtypeStruct((B, NCAND), jnp.float32),
            jax.ShapeDtypeStruct((B, NCAND), jnp.int32),
            jax.ShapeDtypeStruct((B, NCAND), jnp.int32),
            jax.ShapeDtypeStruct((B, NCAND), jnp.int32),
        ],
    )(g.reshape(B, NCAND * C), bids)

    scores, labels, staged = _sc_postprocess(
        vals.reshape(-1), pos.reshape(-1), gr.reshape(-1),
        out_bbox.reshape(B * N * 4 // 128, 128), B, NCAND, C
    )

    img_h = target_sizes[:, 0].astype(jnp.float32)
    img_w = target_sizes[:, 1].astype(jnp.float32)
    scale = jnp.stack([img_w, img_h, img_w, img_h], axis=1)
    scale_rep = jnp.repeat(scale, NCAND, axis=0)
    T = B * NCAND
    boxes = pl.pallas_call(
        _boxes_body,
        in_specs=[
            pl.BlockSpec((T, 128), lambda: (0, 0)),
            pl.BlockSpec((T, 1), lambda: (0, 0)),
            pl.BlockSpec((T, 4), lambda: (0, 0)),
        ],
        out_specs=pl.BlockSpec((T, 4), lambda: (0, 0)),
        out_shape=jax.ShapeDtypeStruct((T, 4), jnp.float32),
    )(staged, gm.reshape(T, 1), scale_rep)

    scores = scores.reshape(B, NCAND)[:, :100]
    labels = labels.reshape(B, NCAND)[:, :100]
    boxes = boxes.reshape(B, NCAND, 4)[:, :100]
    return scores, labels, boxes
